# baseline (speedup 1.0000x reference)
import jax
import jax.numpy as jnp
from jax.experimental import pallas as pl
from jax.experimental.pallas import tpu as pltpu

N = 4096
F = 32
BR = 64


def _layer_kernel(h01, h02, h10, h12, h20, h21,
                  x0, x1, x2, w0, w1, w2, b0, b1, b2,
                  o0, o1, o2):
    prec = jax.lax.Precision.DEFAULT
    u0 = (jnp.dot(h01[...], x1[...], precision=prec)
          + jnp.dot(h02[...], x2[...], precision=prec))
    o0[...] = jnp.dot(u0, w0[...], precision=prec) + b0[...]
    u1 = (jnp.dot(h10[...], x0[...], precision=prec)
          + jnp.dot(h12[...], x2[...], precision=prec))
    o1[...] = jnp.dot(u1, w1[...], precision=prec) + b1[...]
    u2 = (jnp.dot(h20[...], x0[...], precision=prec)
          + jnp.dot(h21[...], x1[...], precision=prec))
    o2[...] = jnp.dot(u2, w2[...], precision=prec) + b2[...]


def _hgnn_layer(H01, H02, H10, H12, H20, H21, x0, x1, x2,
                w0, w1, w2, b0, b1, b2):
    nb = N // BR
    h_spec = pl.BlockSpec((BR, N), lambda r: (r, 0))
    full = pl.BlockSpec((N, F), lambda r: (0, 0))
    w_spec = pl.BlockSpec((F, F), lambda r: (0, 0))
    b_spec = pl.BlockSpec((1, F), lambda r: (0, 0))
    out_spec = pl.BlockSpec((BR, F), lambda r: (r, 0))
    return pl.pallas_call(
        _layer_kernel,
        grid=(nb,),
        in_specs=[h_spec] * 6 + [full] * 3 + [w_spec] * 3 + [b_spec] * 3,
        out_specs=[out_spec] * 3,
        out_shape=[jax.ShapeDtypeStruct((N, F), jnp.float32)] * 3,
        compiler_params=pltpu.CompilerParams(
            dimension_semantics=("arbitrary",),
        ),
    )(H01, H02, H10, H12, H20, H21, x0, x1, x2,
      w0, w1, w2, b0, b1, b2)


def kernel(x0, x1, x2, H01, H02, H10, H12, H20, H21,
           W1_0, b1_0, W1_1, b1_1, W1_2, b1_2,
           W2_0, b2_0, W2_1, b2_1, W2_2, b2_2):
    b1 = [b.reshape(1, F) for b in (b1_0, b1_1, b1_2)]
    b2 = [b.reshape(1, F) for b in (b2_0, b2_1, b2_2)]
    h0, h1, h2 = _hgnn_layer(H01, H02, H10, H12, H20, H21, x0, x1, x2,
                             W1_0, W1_1, W1_2, *b1)
    o0, o1, o2 = _hgnn_layer(H01, H02, H10, H12, H20, H21, h0, h1, h2,
                             W2_0, W2_1, W2_2, *b2)
    return (o0, o1, o2)


# speedup vs baseline: 1.1431x; 1.1431x over previous
import jax
import jax.numpy as jnp
from jax.experimental import pallas as pl
from jax.experimental.pallas import tpu as pltpu

N = 4096
F = 32
BR = 128


def _kernel(h01, h02, h10, h12, h20, h21,
            x0, x1, x2, w10, w11, w12, b10, b11, b12,
            w20, w21, w22, b20, b21, b22,
            o0, o1, o2, s0, s1, s2):
    phase = pl.program_id(0)
    r = pl.program_id(1)
    rows = pl.ds(r * BR, BR)

    def agg(ha, xa, hb, xb, w, b):
        u = jnp.dot(ha[...], xa, preferred_element_type=jnp.float32)
        u += jnp.dot(hb[...], xb, preferred_element_type=jnp.float32)
        return jnp.dot(u, w[...], preferred_element_type=jnp.float32) + b[...]

    @pl.when(phase == 0)
    def _layer1():
        s0[rows, :] = agg(h01, x1[...], h02, x2[...], w10, b10)
        s1[rows, :] = agg(h10, x0[...], h12, x2[...], w11, b11)
        s2[rows, :] = agg(h20, x0[...], h21, x1[...], w12, b12)

    @pl.when(phase == 1)
    def _layer2():
        o0[...] = agg(h01, s1[...], h02, s2[...], w20, b20)
        o1[...] = agg(h10, s0[...], h12, s2[...], w21, b21)
        o2[...] = agg(h20, s0[...], h21, s1[...], w22, b22)


def kernel(x0, x1, x2, H01, H02, H10, H12, H20, H21,
           W1_0, b1_0, W1_1, b1_1, W1_2, b1_2,
           W2_0, b2_0, W2_1, b2_1, W2_2, b2_2):
    nb = N // BR
    h_spec = pl.BlockSpec((BR, N), lambda p, r: (r, 0))
    full = pl.BlockSpec((N, F), lambda p, r: (0, 0))
    w_spec = pl.BlockSpec((F, F), lambda p, r: (0, 0))
    b_spec = pl.BlockSpec((1, F), lambda p, r: (0, 0))
    out_spec = pl.BlockSpec((BR, F), lambda p, r: (r, 0))
    outs = pl.pallas_call(
        _kernel,
        grid=(2, nb),
        in_specs=[h_spec] * 6 + [full] * 3 + [w_spec] * 3 + [b_spec] * 3
                 + [w_spec] * 3 + [b_spec] * 3,
        out_specs=[out_spec] * 3,
        out_shape=[jax.ShapeDtypeStruct((N, F), jnp.float32)] * 3,
        scratch_shapes=[pltpu.VMEM((N, F), jnp.float32)] * 3,
        compiler_params=pltpu.CompilerParams(
            dimension_semantics=("arbitrary", "arbitrary"),
        ),
    )(H01, H02, H10, H12, H20, H21, x0, x1, x2,
      W1_0, W1_1, W1_2,
      b1_0.reshape(1, F), b1_1.reshape(1, F), b1_2.reshape(1, F),
      W2_0, W2_1, W2_2,
      b2_0.reshape(1, F), b2_1.reshape(1, F), b2_2.reshape(1, F))
    return tuple(outs)
